# baseline (speedup 1.0000x reference)
import functools

import jax
import jax.numpy as jnp
from jax import lax
from jax.experimental import pallas as pl
from jax.experimental.pallas import tpu as pltpu
from jax.experimental.pallas import tpu_sc as plsc

_H = 200
_BT = 128
_DIM = 64
_NW = 32
_NSLOT = 4
_NBLK = (_H * _BT) // _NW


def _emb_body(idx_hbm, table_hbm, out_hbm,
              idx_v, gb0, gb1, gb2, gb3, tb0, tb1, tb2, tb3,
              isem, gsem, osem):
    gbufs = (gb0, gb1, gb2, gb3)
    tbufs = (tb0, tb1, tb2, tb3)
    wid = lax.axis_index("s") * 2 + lax.axis_index("c")
    blk0 = wid * _NBLK
    rows = [lax.iota(jnp.int32, 16) + 16 * k for k in range(8)]

    def hb(g):
        blk = blk0 + g
        return blk // _BT, blk % _BT

    def i_copy(g, s):
        h, bt = hb(g)
        return pltpu.make_async_copy(
            idx_hbm.at[h].at[pl.ds(bt * 128, 128)], idx_v.at[s], isem.at[s])

    def g_copy(g, s):
        return pltpu.make_async_copy(
            table_hbm.at[idx_v.at[s]], gbufs[s], gsem.at[s])

    def o_copy(g, s, dh):
        h, bt = hb(g)
        return pltpu.make_async_copy(
            tbufs[s].at[dh], out_hbm.at[h, dh, bt], osem.at[s])

    def transpose(s):
        gb, tb = gbufs[s], tbufs[s]

        def dbody(d, carry):
            cols = jnp.full((16,), d, jnp.int32)
            dh = d // 8
            dl = d % 8
            for b16 in range(8):
                v = plsc.load_gather(gb, [rows[b16], cols])
                tb[dh, dl, pl.ds(b16 * 16, 16)] = v
            return carry

        lax.fori_loop(0, _DIM, dbody, 0)

    def step(g, s, *, wait_i_next, issue_i2, wait_o):
        if wait_i_next:
            i_copy(g + 1, (s + 1) % _NSLOT).wait()
            g_copy(g + 1, (s + 1) % _NSLOT).start()
        if issue_i2:
            i_copy(g + 2, (s + 2) % _NSLOT).start()
        g_copy(g, s).wait()
        if wait_o:
            for dh in range(8):
                o_copy(g - _NSLOT, s, dh).wait()
        transpose(s)
        for dh in range(8):
            o_copy(g, s, dh).start()

    i_copy(0, 0).start()
    i_copy(1, 1).start()
    i_copy(0, 0).wait()
    g_copy(0, 0).start()

    for k in range(_NSLOT):
        step(k, k, wait_i_next=True, issue_i2=True, wait_o=False)

    def body(G, carry):
        g0 = G * _NSLOT
        for k in range(_NSLOT):
            step(g0 + k, k, wait_i_next=True, issue_i2=True, wait_o=True)
        return carry

    lax.fori_loop(1, _NBLK // _NSLOT - 1, body, 0)

    g0 = _NBLK - _NSLOT
    step(g0 + 0, 0, wait_i_next=True, issue_i2=True, wait_o=True)
    step(g0 + 1, 1, wait_i_next=True, issue_i2=True, wait_o=True)
    step(g0 + 2, 2, wait_i_next=True, issue_i2=False, wait_o=True)
    step(g0 + 3, 3, wait_i_next=False, issue_i2=False, wait_o=True)

    for s in range(_NSLOT):
        for dh in range(8):
            o_copy(_NBLK - _NSLOT + s, s, dh).wait()


def kernel(token_ids, weight):
    tid_t = jnp.transpose(token_ids)

    mesh = plsc.VectorSubcoreMesh(core_axis_name="c", subcore_axis_name="s")
    emb = functools.partial(
        pl.kernel,
        mesh=mesh,
        out_type=jax.ShapeDtypeStruct((_H, 8, _BT, 8, 128), jnp.float32),
        scratch_types=[
            pltpu.VMEM((_NSLOT, 128), jnp.int32),
            pltpu.VMEM((128, _DIM), jnp.float32),
            pltpu.VMEM((128, _DIM), jnp.float32),
            pltpu.VMEM((128, _DIM), jnp.float32),
            pltpu.VMEM((128, _DIM), jnp.float32),
            pltpu.VMEM((8, 8, 128), jnp.float32),
            pltpu.VMEM((8, 8, 128), jnp.float32),
            pltpu.VMEM((8, 8, 128), jnp.float32),
            pltpu.VMEM((8, 8, 128), jnp.float32),
            pltpu.SemaphoreType.DMA((_NSLOT,)),
            pltpu.SemaphoreType.DMA((_NSLOT,)),
            pltpu.SemaphoreType.DMA((_NSLOT,)),
        ],
        compiler_params=pltpu.CompilerParams(
            use_tc_tiling_on_sc=False, needs_layout_passes=False),
    )(_emb_body)
    out5 = emb(tid_t, weight)
    return out5.transpose(2, 4, 0, 1, 3).reshape(16384, _H, _DIM)
